# Initial kernel scaffold; baseline (speedup 1.0000x reference)
import jax
import jax.numpy as jnp
from jax.experimental import pallas as pl


def kernel(x, edge_index, W1l, b1, W1r, W2l, b2, W2r):
    raise NotImplementedError("write your pallas kernel here")


# speedup vs baseline: 4.8721x; 4.8721x over previous
import jax
import jax.numpy as jnp
from jax import lax
from jax.experimental import pallas as pl
from jax.experimental.pallas import tpu as pltpu
from jax.experimental.pallas import tpu_sc as plsc

N_NODES = 10000
N_PAD = 10240
N_CORES = 2
N_SUB = 16
N_TILES = N_CORES * N_SUB
CHUNK = 128
T_CHUNKS = 79
E_PAD = N_TILES * T_CHUNKS * CHUNK
RPT = N_PAD // N_SUB


def _make_segsum():
  mesh = plsc.VectorSubcoreMesh(core_axis_name="c", subcore_axis_name="s")
  out_type = [jax.ShapeDtypeStruct((N_CORES, N_PAD, 128), jnp.float32)]
  scratch = [
      pltpu.VMEM((CHUNK,), jnp.int32),
      pltpu.VMEM((CHUNK,), jnp.int32),
      pltpu.VMEM((CHUNK, 128), jnp.float32),
      pltpu.VMEM((16, 128), jnp.float32),
      pltpu.VMEM_SHARED((N_PAD, 128), jnp.float32),
      pltpu.SemaphoreType.DMA,
  ]

  def body(feat, srcg, dstg, out, sidx, didx, rows, zbuf, acc, sem):
    c = lax.axis_index("c")
    s = lax.axis_index("s")
    wid = c * N_SUB + s

    z16 = jnp.zeros((16,), jnp.float32)
    for i in range(16):
      for j in range(128 // 16):
        zbuf[i, pl.ds(j * 16, 16)] = z16

    base = s * RPT

    def zbody(i, carry):
      pltpu.sync_copy(zbuf, acc.at[pl.ds(base + i * 16, 16)])
      return carry

    lax.fori_loop(0, RPT // 16, zbody, 0)
    plsc.subcore_barrier()

    def ebody(t, carry):
      pltpu.sync_copy(srcg.at[wid, t], sidx)
      pltpu.sync_copy(dstg.at[wid, t], didx)
      pltpu.async_copy(feat.at[sidx], rows, sem).wait()
      pltpu.sync_copy(rows, acc.at[didx], add=True)
      return carry

    lax.fori_loop(0, T_CHUNKS, ebody, 0)
    plsc.subcore_barrier()

    pltpu.sync_copy(acc.at[pl.ds(base, RPT)], out.at[c, pl.ds(base, RPT)])

  return pl.kernel(body, out_type=out_type, mesh=mesh, scratch_types=scratch)


def _make_deghist():
  mesh = plsc.VectorSubcoreMesh(core_axis_name="c", subcore_axis_name="s")

  def body(dstg, dego, didx, hist):
    c = lax.axis_index("c")
    s = lax.axis_index("s")
    wid = c * N_SUB + s
    z16 = jnp.zeros((16,), jnp.float32)
    o16 = jnp.ones((16,), jnp.float32)

    def zb(i, carry):
      hist[pl.ds(i * 16, 16)] = z16
      return carry

    lax.fori_loop(0, N_PAD // 16, zb, 0)

    def eb(t, carry):
      pltpu.sync_copy(dstg.at[wid, t], didx)
      for k in range(CHUNK // 16):
        plsc.addupdate_scatter(hist, [didx[pl.ds(k * 16, 16)]], o16)
      return carry

    lax.fori_loop(0, T_CHUNKS, eb, 0)
    pltpu.sync_copy(hist, dego.at[wid])

  return pl.kernel(
      body,
      out_type=[jax.ShapeDtypeStruct((N_TILES, N_PAD), jnp.float32)],
      mesh=mesh,
      compiler_params=pltpu.CompilerParams(needs_layout_passes=False),
      scratch_types=[pltpu.VMEM((CHUNK,), jnp.int32),
                     pltpu.VMEM((N_PAD,), jnp.float32)])


_segsum = _make_segsum()
_deghist = _make_deghist()

_R = 1024


def _tc1_body(p0, p1, dg, xb, w1l, b1, w1r, w2l, w2r, b2, g_out, r_out):
  deg = jnp.sum(dg[...], axis=1, keepdims=True)
  rdeg = 1.0 / jnp.maximum(deg, 1.0)
  mean = (p0[...] + p1[...]) * rdeg
  h = jnp.dot(mean, w1l[...], preferred_element_type=jnp.float32)
  h = h + jnp.dot(xb[...], w1r[...], preferred_element_type=jnp.float32)
  h = jnp.maximum(h + b1[...], 0.0)
  g_out[...] = jnp.dot(h, w2l[...], preferred_element_type=jnp.float32)
  r_out[...] = jnp.dot(h, w2r[...], preferred_element_type=jnp.float32) + b2[...]


def _tc2_body(q0, q1, dg, r, o):
  deg = jnp.sum(dg[...], axis=1, keepdims=True)
  rdeg = 1.0 / jnp.maximum(deg, 1.0)
  o[...] = (q0[...] + q1[...]) * rdeg + r[...]


_tc1 = pl.pallas_call(
    _tc1_body,
    grid=(N_PAD // _R,),
    in_specs=[
        pl.BlockSpec((_R, 128), lambda i: (i, 0)),
        pl.BlockSpec((_R, 128), lambda i: (i, 0)),
        pl.BlockSpec((_R, N_TILES), lambda i: (i, 0)),
        pl.BlockSpec((_R, 128), lambda i: (i, 0)),
        pl.BlockSpec((128, 256), lambda i: (0, 0)),
        pl.BlockSpec((1, 256), lambda i: (0, 0)),
        pl.BlockSpec((128, 256), lambda i: (0, 0)),
        pl.BlockSpec((256, 128), lambda i: (0, 0)),
        pl.BlockSpec((256, 128), lambda i: (0, 0)),
        pl.BlockSpec((1, 128), lambda i: (0, 0)),
    ],
    out_specs=[
        pl.BlockSpec((_R, 128), lambda i: (i, 0)),
        pl.BlockSpec((_R, 128), lambda i: (i, 0)),
    ],
    out_shape=[
        jax.ShapeDtypeStruct((N_PAD, 128), jnp.float32),
        jax.ShapeDtypeStruct((N_PAD, 128), jnp.float32),
    ],
)

_tc2 = pl.pallas_call(
    _tc2_body,
    grid=(N_PAD // _R,),
    in_specs=[
        pl.BlockSpec((_R, 128), lambda i: (i, 0)),
        pl.BlockSpec((_R, 128), lambda i: (i, 0)),
        pl.BlockSpec((_R, N_TILES), lambda i: (i, 0)),
        pl.BlockSpec((_R, 128), lambda i: (i, 0)),
    ],
    out_specs=pl.BlockSpec((_R, 128), lambda i: (i, 0)),
    out_shape=jax.ShapeDtypeStruct((N_PAD, 128), jnp.float32),
)


def kernel(x, edge_index, W1l, b1, W1r, W2l, b2, W2r):
  src = edge_index[0].astype(jnp.int32)
  dst = edge_index[1].astype(jnp.int32)
  e = src.shape[0]
  pad = E_PAD - e
  srcg = jnp.concatenate([src, jnp.zeros((pad,), jnp.int32)]).reshape(
      N_TILES, T_CHUNKS, CHUNK)
  dstg = jnp.concatenate([dst, jnp.full((pad,), N_NODES, jnp.int32)]).reshape(
      N_TILES, T_CHUNKS, CHUNK)
  xp = jnp.pad(x, ((0, N_PAD - x.shape[0]), (0, 0)))
  (p,) = _segsum(xp, srcg, dstg)
  (dego,) = _deghist(dstg)
  degs = dego.T
  g, r = _tc1(p[0], p[1], degs, xp, W1l.T, b1.reshape(1, -1), W1r.T,
              W2l.T, W2r.T, b2.reshape(1, -1))
  (q,) = _segsum(g, srcg, dstg)
  out = _tc2(q[0], q[1], degs, r)
  return out[:N_NODES]
